# baseline (speedup 1.0000x reference)
import jax
import jax.numpy as jnp
from jax import lax
from jax.experimental import pallas as pl
from jax.experimental.pallas import tpu as pltpu

_SKEL_ITERS = 10
_DT_ITERS = 20
_NROWS = 16

_ONES = 0xFFFFFFFF
_ZERO = 0


def _shift_rows_down(x):
    return jnp.concatenate([x[:1], x[:-1]], axis=0)


def _shift_rows_up(x):
    return jnp.concatenate([x[1:], x[-1:]], axis=0)


def _shift_cols_right(x):
    return jnp.concatenate([x[:, :1], x[:, :-1]], axis=1)


def _shift_cols_left(x):
    return jnp.concatenate([x[:, 1:], x[:, -1:]], axis=1)


def _max3v(x):
    return jnp.maximum(jnp.maximum(_shift_rows_down(x), x), _shift_rows_up(x))


def _max3h(x):
    return jnp.maximum(jnp.maximum(_shift_cols_right(x), x), _shift_cols_left(x))


def _min3v(x):
    return jnp.minimum(jnp.minimum(_shift_rows_down(x), x), _shift_rows_up(x))


def _min3h(x):
    return jnp.minimum(jnp.minimum(_shift_cols_right(x), x), _shift_cols_left(x))


def _dilate3(x):
    return _max3h(_max3v(x))


def _soft_erode(x):
    return jnp.minimum(_min3v(x), _min3h(x))


def _relu(x):
    return jnp.maximum(x, 0.0)


def _soft_skeleton_bf16(img_f32):
    img = img_f32.astype(jnp.bfloat16)

    def body(_, carry):
        e, skel = carry
        e_next = _soft_erode(e)
        delta = _relu(e - _dilate3(e_next))
        return e_next, skel + _relu(delta - skel * delta)

    _, skel = jax.lax.fori_loop(0, _SKEL_ITERS + 1, body,
                                (img, jnp.zeros_like(img)))
    return skel.astype(jnp.float32)


def _pack_matrix(h):
    nq = h // 32
    k = lax.broadcasted_iota(jnp.int32, (2 * nq, h), 0)
    r = lax.broadcasted_iota(jnp.int32, (2 * nq, h), 1)
    cond = ((r >> 5) == (k % nq)) & (((r >> 4) & 1) == (k // nq))
    pow2 = jnp.left_shift(jnp.int32(1), r & 15).astype(jnp.float32)
    return jnp.where(cond, pow2, 0.0)


def _pack_bits_mxu(a, m_f32):
    nq = m_f32.shape[0] // 32
    out = jnp.dot(a, m_f32, preferred_element_type=jnp.float32)
    u = out.astype(jnp.int32).astype(jnp.uint32)
    return u[:nq] | (u[nq:] << 16)


def _unpack_matrix(h):
    r = lax.broadcasted_iota(jnp.int32, (h, h // 8), 0)
    k = lax.broadcasted_iota(jnp.int32, (h, h // 8), 1)
    return ((r >> 3) == k).astype(jnp.float32)


def _unpack_f32_mxu(rmat, bits_list):
    nq, w_ = bits_list[0].shape
    wfs = []
    for bits in bits_list:
        planes = [((bits >> (8 * j)) & 0xFF).astype(jnp.int32)
                  .astype(jnp.float32) for j in range(4)]
        wfs.append(jnp.stack(planes, axis=1).reshape(4 * nq, w_))
    wf = jnp.concatenate(wfs, axis=1) if len(wfs) > 1 else wfs[0]
    rep = jnp.dot(rmat, wf, preferred_element_type=jnp.float32)
    shift = lax.broadcasted_iota(jnp.uint32, rep.shape, 0) & 7
    pow2m = lax.bitcast_convert_type((127 - shift) << 23, jnp.float32)
    q = jnp.floor(rep * pow2m)
    ones = q - 2.0 * jnp.floor(q * 0.5)
    return [ones[:, i * w_:(i + 1) * w_] for i in range(len(bits_list))]


def _next_words(w, fill):
    return jnp.concatenate(
        [w[1:], jnp.full((1, w.shape[1]), fill, w.dtype)], axis=0)


def _prev_words(w, fill):
    return jnp.concatenate(
        [jnp.full((1, w.shape[1]), fill, w.dtype), w[:-1]], axis=0)


def _col_next(w, fill):
    return jnp.concatenate(
        [w[:, 1:], jnp.full((w.shape[0], 1), fill, w.dtype)], axis=1)


def _col_prev(w, fill):
    return jnp.concatenate(
        [jnp.full((w.shape[0], 1), fill, w.dtype), w[:, :-1]], axis=1)


def _row_nbrs_and(w):
    up = (w >> 1) | (_next_words(w, _ONES) << 31)
    dn = (w << 1) | (_prev_words(w, _ONES) >> 31)
    return up & dn


def _erode_packed(w):
    ev = w & _row_nbrs_and(w)
    return ev & _col_next(ev, _ONES) & _col_prev(ev, _ONES)


def _dilate_packed(w):
    dv = (w | (w >> 1) | (_next_words(w, _ZERO) << 31)
            | (w << 1) | (_prev_words(w, _ZERO) >> 31))
    return dv | _col_next(dv, _ZERO) | _col_prev(dv, _ZERO)


def _soft_erode_packed(w):
    return w & _row_nbrs_and(w) & _col_next(w, _ONES) & _col_prev(w, _ONES)


def _soft_skeleton_packed(t_bits):
    def cond(st):
        i, _, _, changed = st
        return jnp.logical_and(i < _SKEL_ITERS + 1, changed)

    def body(st):
        i, e, skel, _ = st
        e_next = _soft_erode_packed(e)
        delta = e & ~_dilate_packed(e_next)
        return i + 1, e_next, skel | delta, jnp.any(e_next != e)

    _, _, skel, _ = lax.while_loop(
        cond, body,
        (jnp.int32(0), t_bits, jnp.zeros_like(t_bits), jnp.bool_(True)))
    return skel


def _popcount_sum(bits):
    return jnp.sum(lax.population_count(bits).astype(jnp.float32))


def _dt_weighted_popsum(e_bits, w_bits):
    def cond(st):
        j, _, _, changed = st
        return jnp.logical_and(j < _DT_ITERS, changed)

    def body(st):
        j, e, acc, _ = st
        acc = acc + lax.population_count(e & w_bits)
        e2 = _erode_packed(e)
        return j + 1, e2, acc, jnp.any(e2 != e)

    j, e, acc, _ = lax.while_loop(
        cond, body,
        (jnp.int32(0), e_bits, jnp.zeros(e_bits.shape, jnp.uint32),
         jnp.bool_(True)))
    tail = (_DT_ITERS - j).astype(jnp.float32) * jnp.sum(
        lax.population_count(e & w_bits).astype(jnp.float32))
    return jnp.sum(acc.astype(jnp.float32)) + tail


def _loss_body(pred_ref, tgt_ref, out_ref):
    x = pred_ref[0, 0]
    h = x.shape[0]
    t = tgt_ref[0, 0].astype(jnp.float32)
    p = jax.nn.sigmoid(x)
    bce = _relu(x) - x * t + jnp.log1p(jnp.exp(-jnp.abs(x)))

    amat = _pack_matrix(h)
    t_bits = _pack_bits_mxu(amat, t)
    pb_bits = _pack_bits_mxu(amat, (p > 0.5).astype(jnp.float32))

    boundary_bits = _dilate_packed(t_bits) & ~_erode_packed(t_bits)
    skel_p = _soft_skeleton_bf16(p)
    skel_t_bits = _soft_skeleton_packed(t_bits)
    boundary_f, skel_t = _unpack_f32_mxu(
        _unpack_matrix(h), [boundary_bits, skel_t_bits])

    s_bce = jnp.sum(bce)
    s_wbce = s_bce + 3.0 * jnp.sum(boundary_f * bce)

    dt_fwd = _dt_weighted_popsum(~pb_bits, t_bits)
    dt_bwd = _dt_weighted_popsum(~t_bits, pb_bits)

    scalars = [
        s_bce, jnp.sum(p * t), jnp.sum(p), _popcount_sum(t_bits),
        s_wbce, jnp.sum(skel_p * t), jnp.sum(skel_p),
        jnp.sum(skel_t * p), _popcount_sum(skel_t_bits),
        _popcount_sum(pb_bits), dt_fwd, dt_bwd,
    ]
    scalars += [jnp.float32(0.0)] * (_NROWS - len(scalars))
    rows = [jnp.full((1, 128), s, jnp.float32) for s in scalars]
    out_ref[0] = jnp.concatenate(rows, axis=0)


def kernel(pred, target):
    B, C, H, W = pred.shape
    partials = pl.pallas_call(
        _loss_body,
        grid=(B,),
        in_specs=[
            pl.BlockSpec((1, C, H, W), lambda b: (b, 0, 0, 0)),
            pl.BlockSpec((1, C, H, W), lambda b: (b, 0, 0, 0)),
        ],
        out_specs=pl.BlockSpec((1, _NROWS, 128), lambda b: (b, 0, 0)),
        out_shape=jax.ShapeDtypeStruct((B, _NROWS, 128), jnp.float32),
        compiler_params=pltpu.CompilerParams(
            dimension_semantics=("parallel",),
        ),
    )(pred, target)

    s = jnp.sum(partials[:, :, 0], axis=0)
    (s_bce, s_pt, s_p, s_t, s_wbce, s_spt, s_sp, s_stp, s_st, s_pb,
     dt_f, dt_b) = (s[i] for i in range(12))
    n = jnp.float32(pred.size)
    smooth = 1.0

    loss_bce = s_bce / n
    loss_dice = 1.0 - (2.0 * s_pt + smooth) / (s_p + s_t + smooth)
    fp = s_p - s_pt
    fn = s_t - s_pt
    tversky = (s_pt + smooth) / (s_pt + 0.3 * fp + 0.7 * fn + smooth)
    loss_ft = (1.0 - tversky) ** 1.33
    loss_boundary = s_wbce / n
    eps = 1.0
    tprec = (s_spt + eps) / (s_sp + eps)
    tsens = (s_stp + eps) / (s_st + eps)
    loss_cldice = 1.0 - 2.0 * tprec * tsens / (tprec + tsens)
    hsm = 1e-6
    hd_fwd = (dt_f + hsm) / (s_t + hsm)
    hd_bwd = (dt_b + hsm) / (s_pb + hsm)
    loss_hd = 0.5 * (hd_fwd + hd_bwd)

    return (0.2 * loss_bce + 0.2 * loss_dice + 0.2 * loss_cldice
            + 0.1 * loss_hd + 0.1 * loss_boundary + 0.2 * loss_ft)


# speedup vs baseline: 1.1250x; 1.1250x over previous
import jax
import jax.numpy as jnp
from jax import lax
from jax.experimental import pallas as pl
from jax.experimental.pallas import tpu as pltpu

_SKEL_ITERS = 10
_DT_ITERS = 20
_NROWS = 16
_BB = 2

_ONES = 0xFFFFFFFF
_ZERO = 0


def _shift_rows_down(x):
    return jnp.concatenate([x[..., :1, :], x[..., :-1, :]], axis=-2)


def _shift_rows_up(x):
    return jnp.concatenate([x[..., 1:, :], x[..., -1:, :]], axis=-2)


def _shift_cols_right(x):
    return jnp.concatenate([x[..., :, :1], x[..., :, :-1]], axis=-1)


def _shift_cols_left(x):
    return jnp.concatenate([x[..., :, 1:], x[..., :, -1:]], axis=-1)


def _max3v(x):
    return jnp.maximum(jnp.maximum(_shift_rows_down(x), x), _shift_rows_up(x))


def _max3h(x):
    return jnp.maximum(jnp.maximum(_shift_cols_right(x), x), _shift_cols_left(x))


def _min3v(x):
    return jnp.minimum(jnp.minimum(_shift_rows_down(x), x), _shift_rows_up(x))


def _min3h(x):
    return jnp.minimum(jnp.minimum(_shift_cols_right(x), x), _shift_cols_left(x))


def _dilate3(x):
    return _max3h(_max3v(x))


def _soft_erode(x):
    return jnp.minimum(_min3v(x), _min3h(x))


def _relu(x):
    return jnp.maximum(x, 0.0)


def _soft_skeleton_bf16(img_f32):
    img = img_f32.astype(jnp.bfloat16)

    def body(_, carry):
        e, skel = carry
        e_next = _soft_erode(e)
        delta = _relu(e - _dilate3(e_next))
        return e_next, skel + _relu(delta - skel * delta)

    _, skel = jax.lax.fori_loop(0, _SKEL_ITERS + 1, body,
                                (img, jnp.zeros_like(img)))
    return skel.astype(jnp.float32)


def _pack_matrix(h):
    nq = h // 32
    k = lax.broadcasted_iota(jnp.int32, (2 * nq, h), 0)
    r = lax.broadcasted_iota(jnp.int32, (2 * nq, h), 1)
    cond = ((r >> 5) == (k % nq)) & (((r >> 4) & 1) == (k // nq))
    pow2 = jnp.left_shift(jnp.int32(1), r & 15).astype(jnp.float32)
    return jnp.where(cond, pow2, 0.0)


def _pack_bits_mxu(a, m_f32):
    nq = m_f32.shape[0] // 32
    out = jnp.dot(a, m_f32, preferred_element_type=jnp.float32)
    u = out.astype(jnp.int32).astype(jnp.uint32)
    return u[:nq] | (u[nq:] << 16)


def _unpack_matrix(h):
    r = lax.broadcasted_iota(jnp.int32, (h, h // 8), 0)
    k = lax.broadcasted_iota(jnp.int32, (h, h // 8), 1)
    return ((r >> 3) == k).astype(jnp.float32)


def _unpack_f32_mxu(rmat, bits_list):
    nq, w_ = bits_list[0].shape
    wfs = []
    for bits in bits_list:
        planes = [((bits >> (8 * j)) & 0xFF).astype(jnp.int32)
                  .astype(jnp.float32) for j in range(4)]
        wfs.append(jnp.stack(planes, axis=1).reshape(4 * nq, w_))
    wf = jnp.concatenate(wfs, axis=1) if len(wfs) > 1 else wfs[0]
    rep = jnp.dot(rmat, wf, preferred_element_type=jnp.float32)
    shift = lax.broadcasted_iota(jnp.uint32, rep.shape, 0) & 7
    pow2m = lax.bitcast_convert_type((127 - shift) << 23, jnp.float32)
    q = jnp.floor(rep * pow2m)
    ones = q - 2.0 * jnp.floor(q * 0.5)
    return [ones[:, i * w_:(i + 1) * w_] for i in range(len(bits_list))]


def _next_words(w, fill):
    f = jnp.full(w.shape[:-2] + (1, w.shape[-1]), fill, w.dtype)
    return jnp.concatenate([w[..., 1:, :], f], axis=-2)


def _prev_words(w, fill):
    f = jnp.full(w.shape[:-2] + (1, w.shape[-1]), fill, w.dtype)
    return jnp.concatenate([f, w[..., :-1, :]], axis=-2)


def _col_next(w, fill):
    f = jnp.full(w.shape[:-1] + (1,), fill, w.dtype)
    return jnp.concatenate([w[..., :, 1:], f], axis=-1)


def _col_prev(w, fill):
    f = jnp.full(w.shape[:-1] + (1,), fill, w.dtype)
    return jnp.concatenate([f, w[..., :, :-1]], axis=-1)


def _row_nbrs_and(w):
    up = (w >> 1) | (_next_words(w, _ONES) << 31)
    dn = (w << 1) | (_prev_words(w, _ONES) >> 31)
    return up & dn


def _erode_packed(w):
    ev = w & _row_nbrs_and(w)
    return ev & _col_next(ev, _ONES) & _col_prev(ev, _ONES)


def _dilate_packed(w):
    dv = (w | (w >> 1) | (_next_words(w, _ZERO) << 31)
            | (w << 1) | (_prev_words(w, _ZERO) >> 31))
    return dv | _col_next(dv, _ZERO) | _col_prev(dv, _ZERO)


def _soft_erode_packed(w):
    return w & _row_nbrs_and(w) & _col_next(w, _ONES) & _col_prev(w, _ONES)


def _soft_skeleton_packed(t_bits):
    def cond(st):
        i, _, _, changed = st
        return jnp.logical_and(i < _SKEL_ITERS + 1, changed)

    def body(st):
        i, e, skel, _ = st
        e_next = _soft_erode_packed(e)
        delta = e & ~_dilate_packed(e_next)
        return i + 1, e_next, skel | delta, jnp.any(e_next != e)

    _, _, skel, _ = lax.while_loop(
        cond, body,
        (jnp.int32(0), t_bits, jnp.zeros_like(t_bits), jnp.bool_(True)))
    return skel


def _popcount_sum2(bits):
    return jnp.sum(lax.population_count(bits).astype(jnp.float32),
                   axis=(-2, -1))


def _dt_weighted_popsums(e_bits, w_bits):
    def cond(st):
        j, _, _, changed = st
        return jnp.logical_and(j < _DT_ITERS, changed)

    def body(st):
        j, e, acc, _ = st
        acc = acc + lax.population_count(e & w_bits)
        e2 = _erode_packed(e)
        return j + 1, e2, acc, jnp.any(e2 != e)

    j, e, acc, _ = lax.while_loop(
        cond, body,
        (jnp.int32(0), e_bits, jnp.zeros(e_bits.shape, jnp.uint32),
         jnp.bool_(True)))
    tail = (_DT_ITERS - j).astype(jnp.float32) * jnp.sum(
        lax.population_count(e & w_bits).astype(jnp.float32), axis=(-2, -1))
    return jnp.sum(acc.astype(jnp.float32), axis=(-2, -1)) + tail


def _loss_body(pred_ref, tgt_ref, out_ref):
    x = pred_ref[:, 0]
    h = x.shape[1]
    t = tgt_ref[:, 0].astype(jnp.float32)
    p = jax.nn.sigmoid(x)
    bce = _relu(x) - x * t + jnp.log1p(jnp.exp(-jnp.abs(x)))

    amat = _pack_matrix(h)
    pbf = (p > 0.5).astype(jnp.float32)
    t_bits = jnp.stack([_pack_bits_mxu(amat, t[i]) for i in range(_BB)])
    pb_bits = jnp.stack([_pack_bits_mxu(amat, pbf[i]) for i in range(_BB)])

    boundary_bits = _dilate_packed(t_bits) & ~_erode_packed(t_bits)
    skel_p = _soft_skeleton_bf16(p)
    skel_t_bits = _soft_skeleton_packed(t_bits)
    unpacked = _unpack_f32_mxu(
        _unpack_matrix(h),
        [boundary_bits[i] for i in range(_BB)]
        + [skel_t_bits[i] for i in range(_BB)])
    boundary_f = jnp.stack(unpacked[:_BB])
    skel_t = jnp.stack(unpacked[_BB:])

    dt_sums = _dt_weighted_popsums(
        jnp.stack([~pb_bits, ~t_bits]),
        jnp.stack([t_bits, pb_bits]))
    dt_fwd, dt_bwd = dt_sums[0], dt_sums[1]

    ax = (-2, -1)
    s_bce = jnp.sum(bce, ax)
    scalars = [
        s_bce, jnp.sum(p * t, ax), jnp.sum(p, ax), _popcount_sum2(t_bits),
        s_bce + 3.0 * jnp.sum(boundary_f * bce, ax),
        jnp.sum(skel_p * t, ax), jnp.sum(skel_p, ax),
        jnp.sum(skel_t * p, ax), _popcount_sum2(skel_t_bits),
        _popcount_sum2(pb_bits), dt_fwd, dt_bwd,
    ]
    scalars += [jnp.zeros(_BB, jnp.float32)] * (_NROWS - len(scalars))
    for i in range(_BB):
        rows = [jnp.full((1, 128), s[i], jnp.float32) for s in scalars]
        out_ref[i] = jnp.concatenate(rows, axis=0)


def kernel(pred, target):
    B, C, H, W = pred.shape
    partials = pl.pallas_call(
        _loss_body,
        grid=(B // _BB,),
        in_specs=[
            pl.BlockSpec((_BB, C, H, W), lambda b: (b, 0, 0, 0)),
            pl.BlockSpec((_BB, C, H, W), lambda b: (b, 0, 0, 0)),
        ],
        out_specs=pl.BlockSpec((_BB, _NROWS, 128), lambda b: (b, 0, 0)),
        out_shape=jax.ShapeDtypeStruct((B, _NROWS, 128), jnp.float32),
        compiler_params=pltpu.CompilerParams(
            dimension_semantics=("parallel",),
        ),
    )(pred, target)

    s = jnp.sum(partials[:, :, 0], axis=0)
    (s_bce, s_pt, s_p, s_t, s_wbce, s_spt, s_sp, s_stp, s_st, s_pb,
     dt_f, dt_b) = (s[i] for i in range(12))
    n = jnp.float32(pred.size)
    smooth = 1.0

    loss_bce = s_bce / n
    loss_dice = 1.0 - (2.0 * s_pt + smooth) / (s_p + s_t + smooth)
    fp = s_p - s_pt
    fn = s_t - s_pt
    tversky = (s_pt + smooth) / (s_pt + 0.3 * fp + 0.7 * fn + smooth)
    loss_ft = (1.0 - tversky) ** 1.33
    loss_boundary = s_wbce / n
    eps = 1.0
    tprec = (s_spt + eps) / (s_sp + eps)
    tsens = (s_stp + eps) / (s_st + eps)
    loss_cldice = 1.0 - 2.0 * tprec * tsens / (tprec + tsens)
    hsm = 1e-6
    hd_fwd = (dt_f + hsm) / (s_t + hsm)
    hd_bwd = (dt_b + hsm) / (s_pb + hsm)
    loss_hd = 0.5 * (hd_fwd + hd_bwd)

    return (0.2 * loss_bce + 0.2 * loss_dice + 0.2 * loss_cldice
            + 0.1 * loss_hd + 0.1 * loss_boundary + 0.2 * loss_ft)


# speedup vs baseline: 1.1351x; 1.0090x over previous
import jax
import jax.numpy as jnp
from jax import lax
from jax.experimental import pallas as pl
from jax.experimental.pallas import tpu as pltpu

_SKEL_ITERS = 10
_DT_ITERS = 20
_NROWS = 16
_BB = 4

_ONES = 0xFFFFFFFF
_ZERO = 0


def _shift_rows_down(x):
    return jnp.concatenate([x[..., :1, :], x[..., :-1, :]], axis=-2)


def _shift_rows_up(x):
    return jnp.concatenate([x[..., 1:, :], x[..., -1:, :]], axis=-2)


def _shift_cols_right(x):
    return jnp.concatenate([x[..., :, :1], x[..., :, :-1]], axis=-1)


def _shift_cols_left(x):
    return jnp.concatenate([x[..., :, 1:], x[..., :, -1:]], axis=-1)


def _max3v(x):
    return jnp.maximum(jnp.maximum(_shift_rows_down(x), x), _shift_rows_up(x))


def _max3h(x):
    return jnp.maximum(jnp.maximum(_shift_cols_right(x), x), _shift_cols_left(x))


def _min3v(x):
    return jnp.minimum(jnp.minimum(_shift_rows_down(x), x), _shift_rows_up(x))


def _min3h(x):
    return jnp.minimum(jnp.minimum(_shift_cols_right(x), x), _shift_cols_left(x))


def _dilate3(x):
    return _max3h(_max3v(x))


def _soft_erode(x):
    return jnp.minimum(_min3v(x), _min3h(x))


def _relu(x):
    return jnp.maximum(x, 0.0)


def _soft_skeleton_bf16(img_f32):
    img = img_f32.astype(jnp.bfloat16)

    def body(_, carry):
        e, skel = carry
        e_next = _soft_erode(e)
        delta = _relu(e - _dilate3(e_next))
        return e_next, skel + _relu(delta - skel * delta)

    _, skel = jax.lax.fori_loop(0, _SKEL_ITERS + 1, body,
                                (img, jnp.zeros_like(img)))
    return skel.astype(jnp.float32)


def _pack_matrix(h):
    nq = h // 32
    k = lax.broadcasted_iota(jnp.int32, (2 * nq, h), 0)
    r = lax.broadcasted_iota(jnp.int32, (2 * nq, h), 1)
    cond = ((r >> 5) == (k % nq)) & (((r >> 4) & 1) == (k // nq))
    pow2 = jnp.left_shift(jnp.int32(1), r & 15).astype(jnp.float32)
    return jnp.where(cond, pow2, 0.0)


def _pack_bits_mxu(a, m_f32):
    nq = m_f32.shape[0] // 32
    out = jnp.dot(a, m_f32, preferred_element_type=jnp.float32)
    u = out.astype(jnp.int32).astype(jnp.uint32)
    return u[:nq] | (u[nq:] << 16)


def _unpack_matrix(h):
    r = lax.broadcasted_iota(jnp.int32, (h, h // 8), 0)
    k = lax.broadcasted_iota(jnp.int32, (h, h // 8), 1)
    return ((r >> 3) == k).astype(jnp.float32)


def _unpack_f32_mxu(rmat, bits_list):
    nq, w_ = bits_list[0].shape
    wfs = []
    for bits in bits_list:
        planes = [((bits >> (8 * j)) & 0xFF).astype(jnp.int32)
                  .astype(jnp.float32) for j in range(4)]
        wfs.append(jnp.stack(planes, axis=1).reshape(4 * nq, w_))
    wf = jnp.concatenate(wfs, axis=1) if len(wfs) > 1 else wfs[0]
    rep = jnp.dot(rmat, wf, preferred_element_type=jnp.float32)
    shift = lax.broadcasted_iota(jnp.uint32, rep.shape, 0) & 7
    pow2m = lax.bitcast_convert_type((127 - shift) << 23, jnp.float32)
    q = jnp.floor(rep * pow2m)
    ones = q - 2.0 * jnp.floor(q * 0.5)
    return [ones[:, i * w_:(i + 1) * w_] for i in range(len(bits_list))]


def _next_words(w, fill):
    f = jnp.full(w.shape[:-2] + (1, w.shape[-1]), fill, w.dtype)
    return jnp.concatenate([w[..., 1:, :], f], axis=-2)


def _prev_words(w, fill):
    f = jnp.full(w.shape[:-2] + (1, w.shape[-1]), fill, w.dtype)
    return jnp.concatenate([f, w[..., :-1, :]], axis=-2)


def _col_next(w, fill):
    f = jnp.full(w.shape[:-1] + (1,), fill, w.dtype)
    return jnp.concatenate([w[..., :, 1:], f], axis=-1)


def _col_prev(w, fill):
    f = jnp.full(w.shape[:-1] + (1,), fill, w.dtype)
    return jnp.concatenate([f, w[..., :, :-1]], axis=-1)


def _row_nbrs_and(w):
    up = (w >> 1) | (_next_words(w, _ONES) << 31)
    dn = (w << 1) | (_prev_words(w, _ONES) >> 31)
    return up & dn


def _erode_packed(w):
    ev = w & _row_nbrs_and(w)
    return ev & _col_next(ev, _ONES) & _col_prev(ev, _ONES)


def _dilate_packed(w):
    dv = (w | (w >> 1) | (_next_words(w, _ZERO) << 31)
            | (w << 1) | (_prev_words(w, _ZERO) >> 31))
    return dv | _col_next(dv, _ZERO) | _col_prev(dv, _ZERO)


def _soft_erode_packed(w):
    return w & _row_nbrs_and(w) & _col_next(w, _ONES) & _col_prev(w, _ONES)


def _soft_skeleton_packed(t_bits):
    def cond(st):
        i, _, _, changed = st
        return jnp.logical_and(i < _SKEL_ITERS + 1, changed)

    def body(st):
        i, e, skel, _ = st
        e_next = _soft_erode_packed(e)
        delta = e & ~_dilate_packed(e_next)
        return i + 1, e_next, skel | delta, jnp.any(e_next != e)

    _, _, skel, _ = lax.while_loop(
        cond, body,
        (jnp.int32(0), t_bits, jnp.zeros_like(t_bits), jnp.bool_(True)))
    return skel


def _popcount_sum2(bits):
    return jnp.sum(lax.population_count(bits).astype(jnp.float32),
                   axis=(-2, -1))


def _dt_weighted_popsums(e_bits, w_bits):
    def cond(st):
        j, _, _, changed = st
        return jnp.logical_and(j < _DT_ITERS, changed)

    def body(st):
        j, e, acc, _ = st
        acc = acc + lax.population_count(e & w_bits)
        e2 = _erode_packed(e)
        return j + 1, e2, acc, jnp.any(e2 != e)

    j, e, acc, _ = lax.while_loop(
        cond, body,
        (jnp.int32(0), e_bits, jnp.zeros(e_bits.shape, jnp.uint32),
         jnp.bool_(True)))
    tail = (_DT_ITERS - j).astype(jnp.float32) * jnp.sum(
        lax.population_count(e & w_bits).astype(jnp.float32), axis=(-2, -1))
    return jnp.sum(acc.astype(jnp.float32), axis=(-2, -1)) + tail


def _loss_body(pred_ref, tgt_ref, out_ref):
    x = pred_ref[:, 0]
    h = x.shape[1]
    t = tgt_ref[:, 0].astype(jnp.float32)
    p = jax.nn.sigmoid(x)
    bce = _relu(x) - x * t + jnp.log1p(jnp.exp(-jnp.abs(x)))

    amat = _pack_matrix(h)
    pbf = (p > 0.5).astype(jnp.float32)
    t_bits = jnp.stack([_pack_bits_mxu(amat, t[i]) for i in range(_BB)])
    pb_bits = jnp.stack([_pack_bits_mxu(amat, pbf[i]) for i in range(_BB)])

    boundary_bits = _dilate_packed(t_bits) & ~_erode_packed(t_bits)
    skel_p = _soft_skeleton_bf16(p)
    skel_t_bits = _soft_skeleton_packed(t_bits)
    unpacked = _unpack_f32_mxu(
        _unpack_matrix(h),
        [boundary_bits[i] for i in range(_BB)]
        + [skel_t_bits[i] for i in range(_BB)])
    boundary_f = jnp.stack(unpacked[:_BB])
    skel_t = jnp.stack(unpacked[_BB:])

    dt_sums = _dt_weighted_popsums(
        jnp.stack([~pb_bits, ~t_bits]),
        jnp.stack([t_bits, pb_bits]))
    dt_fwd, dt_bwd = dt_sums[0], dt_sums[1]

    ax = (-2, -1)
    s_bce = jnp.sum(bce, ax)
    scalars = [
        s_bce, jnp.sum(p * t, ax), jnp.sum(p, ax), _popcount_sum2(t_bits),
        s_bce + 3.0 * jnp.sum(boundary_f * bce, ax),
        jnp.sum(skel_p * t, ax), jnp.sum(skel_p, ax),
        jnp.sum(skel_t * p, ax), _popcount_sum2(skel_t_bits),
        _popcount_sum2(pb_bits), dt_fwd, dt_bwd,
    ]
    scalars += [jnp.zeros(_BB, jnp.float32)] * (_NROWS - len(scalars))
    for i in range(_BB):
        rows = [jnp.full((1, 128), s[i], jnp.float32) for s in scalars]
        out_ref[i] = jnp.concatenate(rows, axis=0)


def kernel(pred, target):
    B, C, H, W = pred.shape
    partials = pl.pallas_call(
        _loss_body,
        grid=(B // _BB,),
        in_specs=[
            pl.BlockSpec((_BB, C, H, W), lambda b: (b, 0, 0, 0)),
            pl.BlockSpec((_BB, C, H, W), lambda b: (b, 0, 0, 0)),
        ],
        out_specs=pl.BlockSpec((_BB, _NROWS, 128), lambda b: (b, 0, 0)),
        out_shape=jax.ShapeDtypeStruct((B, _NROWS, 128), jnp.float32),
        compiler_params=pltpu.CompilerParams(
            dimension_semantics=("parallel",),
        ),
    )(pred, target)

    s = jnp.sum(partials[:, :, 0], axis=0)
    (s_bce, s_pt, s_p, s_t, s_wbce, s_spt, s_sp, s_stp, s_st, s_pb,
     dt_f, dt_b) = (s[i] for i in range(12))
    n = jnp.float32(pred.size)
    smooth = 1.0

    loss_bce = s_bce / n
    loss_dice = 1.0 - (2.0 * s_pt + smooth) / (s_p + s_t + smooth)
    fp = s_p - s_pt
    fn = s_t - s_pt
    tversky = (s_pt + smooth) / (s_pt + 0.3 * fp + 0.7 * fn + smooth)
    loss_ft = (1.0 - tversky) ** 1.33
    loss_boundary = s_wbce / n
    eps = 1.0
    tprec = (s_spt + eps) / (s_sp + eps)
    tsens = (s_stp + eps) / (s_st + eps)
    loss_cldice = 1.0 - 2.0 * tprec * tsens / (tprec + tsens)
    hsm = 1e-6
    hd_fwd = (dt_f + hsm) / (s_t + hsm)
    hd_bwd = (dt_b + hsm) / (s_pb + hsm)
    loss_hd = 0.5 * (hd_fwd + hd_bwd)

    return (0.2 * loss_bce + 0.2 * loss_dice + 0.2 * loss_cldice
            + 0.1 * loss_hd + 0.1 * loss_boundary + 0.2 * loss_ft)
